# Initial kernel scaffold; baseline (speedup 1.0000x reference)
import jax
import jax.numpy as jnp
from jax import lax
from jax.experimental import pallas as pl
from jax.experimental.pallas import tpu as pltpu
from jax.experimental.pallas import tpu_sc as plsc

N = 10000
E = 320000
NP = 10240
NC = 2
NS = 16
CHUNK = 128
CPT = 80
EPT = CPT * CHUNK
EP = NC * NS * EPT
ROWS_PT = NP // NS

_mesh = plsc.VectorSubcoreMesh(core_axis_name="c", subcore_axis_name="s")


def _zero_rows(ref, width):
    z = jnp.zeros((16,), jnp.float32)

    @pl.loop(0, 128)
    def _(i):
        for k in range(width // 16):
            ref[i, pl.ds(k * 16, 16)] = z


def _deg_body(dst_hbm, out_hbm, dstv, onesb, zb, sdeg):
    c = lax.axis_index("c")
    s = lax.axis_index("s")
    base_chunk = (c * NS + s) * CPT
    pltpu.sync_copy(dst_hbm.at[pl.ds(base_chunk, CPT)], dstv)
    _zero_rows(zb, 16)
    ones = jnp.ones((16,), jnp.float32)

    @pl.loop(0, 128)
    def _(i):
        onesb[i, :] = ones

    for k in range(ROWS_PT // 128):
        pltpu.sync_copy(zb, sdeg.at[pl.ds(s * ROWS_PT + k * 128, 128)])
    plsc.subcore_barrier()

    @pl.loop(0, CPT)
    def _(j):
        pltpu.sync_copy(onesb, sdeg.at[dstv.at[j]], add=True)

    plsc.subcore_barrier()
    for k in range(ROWS_PT // 128):
        r = s * ROWS_PT + k * 128
        pltpu.sync_copy(sdeg.at[pl.ds(r, 128)], out_hbm.at[c, pl.ds(r, 128)])


_deg_kernel = pl.kernel(
    _deg_body,
    out_type=jax.ShapeDtypeStruct((NC, NP, 16), jnp.float32),
    mesh=_mesh,
    scratch_types=[
        pltpu.VMEM((CPT, CHUNK), jnp.int32),
        pltpu.VMEM((128, 16), jnp.float32),
        pltpu.VMEM((128, 16), jnp.float32),
        pltpu.VMEM_SHARED((NP, 16), jnp.float32),
    ],
)


def _scat_body(g_hbm, src_hbm, dst_hbm, out_hbm,
               srcv, dstv, buf0, buf1, zb, acc, sem0, sem1):
    c = lax.axis_index("c")
    s = lax.axis_index("s")
    base_chunk = (c * NS + s) * CPT
    pltpu.sync_copy(src_hbm.at[pl.ds(base_chunk, CPT)], srcv)
    pltpu.sync_copy(dst_hbm.at[pl.ds(base_chunk, CPT)], dstv)
    _zero_rows(zb, 128)
    for k in range(ROWS_PT // 128):
        pltpu.sync_copy(zb, acc.at[pl.ds(s * ROWS_PT + k * 128, 128)])
    plsc.subcore_barrier()

    pltpu.async_copy(g_hbm.at[srcv.at[0]], buf0, sem0)
    pltpu.async_copy(g_hbm.at[srcv.at[1]], buf1, sem1)

    @pl.loop(0, CPT, step=2)
    def _(j0):
        for b, (buf, sem) in enumerate(((buf0, sem0), (buf1, sem1))):
            j = j0 + b
            pltpu.make_async_copy(g_hbm.at[pl.ds(0, 128)], buf, sem).wait()
            pltpu.sync_copy(buf, acc.at[dstv.at[j]], add=True)

            @pl.when(j + 2 < CPT)
            def _():
                pltpu.async_copy(g_hbm.at[srcv.at[j + 2]], buf, sem)

    plsc.subcore_barrier()
    for k in range(ROWS_PT // 128):
        r = s * ROWS_PT + k * 128
        pltpu.sync_copy(acc.at[pl.ds(r, 128)], out_hbm.at[c, pl.ds(r, 128)])


_scat_kernel = pl.kernel(
    _scat_body,
    out_type=jax.ShapeDtypeStruct((NC, NP, 128), jnp.float32),
    mesh=_mesh,
    scratch_types=[
        pltpu.VMEM((CPT, CHUNK), jnp.int32),
        pltpu.VMEM((CPT, CHUNK), jnp.int32),
        pltpu.VMEM((128, 128), jnp.float32),
        pltpu.VMEM((128, 128), jnp.float32),
        pltpu.VMEM((128, 128), jnp.float32),
        pltpu.VMEM_SHARED((NP, 128), jnp.float32),
        pltpu.SemaphoreType.DMA,
        pltpu.SemaphoreType.DMA,
    ],
)


_R = 512
_NB = NP // _R


def _dinv(d0, d1):
    deg = d0[:, 0:1] + d1[:, 0:1] + 1.0
    return lax.rsqrt(jnp.maximum(deg, 1.0))


def _tc0_body(x_ref, w_ref, d0_ref, d1_ref, g_ref):
    h = jnp.dot(x_ref[:], w_ref[:], preferred_element_type=jnp.float32)
    g_ref[:] = h * _dinv(d0_ref[:], d1_ref[:])


def _tcmid_body(a0_ref, a1_ref, gp_ref, d0_ref, d1_ref, w_ref, b_ref, g_ref):
    dinv = _dinv(d0_ref[:], d1_ref[:])
    f = jnp.tanh((a0_ref[:] + a1_ref[:] + gp_ref[:]) * dinv + b_ref[:])
    h = jnp.dot(f, w_ref[:], preferred_element_type=jnp.float32)
    g_ref[:] = h * dinv


def _tcfinal_body(a0_ref, a1_ref, gp_ref, d0_ref, d1_ref, w_ref, b_ref,
                  bo_ref, o_ref):
    dinv = _dinv(d0_ref[:], d1_ref[:])
    f = jnp.tanh((a0_ref[:] + a1_ref[:] + gp_ref[:]) * dinv + b_ref[:])
    o_ref[:] = jnp.dot(f, w_ref[:], preferred_element_type=jnp.float32) \
        + bo_ref[:]


_row_spec = pl.BlockSpec((_R, 128), lambda i: (i, 0))
_deg_spec = pl.BlockSpec((_R, 16), lambda i: (i, 0))
_w_spec = pl.BlockSpec((128, 128), lambda i: (0, 0))
_b_spec = pl.BlockSpec((1, 128), lambda i: (0, 0))
_out_struct = jax.ShapeDtypeStruct((NP, 128), jnp.float32)

_tc0 = pl.pallas_call(
    _tc0_body,
    grid=(_NB,),
    in_specs=[_row_spec, _w_spec, _deg_spec, _deg_spec],
    out_specs=_row_spec,
    out_shape=_out_struct,
)

_tcmid = pl.pallas_call(
    _tcmid_body,
    grid=(_NB,),
    in_specs=[_row_spec, _row_spec, _row_spec, _deg_spec, _deg_spec,
              _w_spec, _b_spec],
    out_specs=_row_spec,
    out_shape=_out_struct,
)

_tcfinal = pl.pallas_call(
    _tcfinal_body,
    grid=(_NB,),
    in_specs=[_row_spec, _row_spec, _row_spec, _deg_spec, _deg_spec,
              _w_spec, _b_spec, _b_spec],
    out_specs=_row_spec,
    out_shape=_out_struct,
)


@jax.jit
def _run(x, edge_index, W0, b0, W1, b1, W2, b2, W3, b3, Wo, bo):
    ei = edge_index.astype(jnp.int32)
    pad = jnp.full((EP - E,), NP - 1, jnp.int32)
    src = jnp.concatenate([ei[0], pad]).reshape(EP // CHUNK, CHUNK)
    dst = jnp.concatenate([ei[1], pad]).reshape(EP // CHUNK, CHUNK)
    xp = jnp.pad(x, ((0, NP - N), (0, 0)))
    b0r = b0.reshape(1, 128)
    b1r = b1.reshape(1, 128)
    b2r = b2.reshape(1, 128)
    b3r = b3.reshape(1, 128)
    bor = bo.reshape(1, 128)

    degs = _deg_kernel(dst)
    d0, d1 = degs[0], degs[1]

    g = _tc0(xp, W0, d0, d1)
    acc = _scat_kernel(g, src, dst)
    g = _tcmid(acc[0], acc[1], g, d0, d1, W1, b0r)
    acc = _scat_kernel(g, src, dst)
    g = _tcmid(acc[0], acc[1], g, d0, d1, W2, b1r)
    acc = _scat_kernel(g, src, dst)
    g = _tcmid(acc[0], acc[1], g, d0, d1, W3, b2r)
    acc = _scat_kernel(g, src, dst)
    out = _tcfinal(acc[0], acc[1], g, d0, d1, Wo, b3r, bor)
    return out[:N]


def kernel(x, edge_index, batch, W0, b0, W1, b1, W2, b2, W3, b3, Wo, bo):
    return _run(x, edge_index, W0, b0, W1, b1, W2, b2, W3, b3, Wo, bo)


# speedup vs baseline: 6.5112x; 6.5112x over previous
import jax
import jax.numpy as jnp
from jax import lax
from jax.experimental import pallas as pl
from jax.experimental.pallas import tpu as pltpu
from jax.experimental.pallas import tpu_sc as plsc

N = 10000
E = 320000
NP = 10240
NC = 2
NS = 16
CHUNK = 64
CPT = 160
EPT = CPT * CHUNK
EP = NC * NS * EPT
ROWS_PT = NP // NS

_mesh = plsc.VectorSubcoreMesh(core_axis_name="c", subcore_axis_name="s")


def _zero_rows(ref, width):
    z = jnp.zeros((16,), jnp.float32)

    @pl.loop(0, 64)
    def _(i):
        for k in range(width // 16):
            ref[i, pl.ds(k * 16, 16)] = z


def _scat_body(g_hbm, src_hbm, dst_hbm, out_hbm,
               srcv, dstv, buf0, buf1, acc, sem0, sem1):
    c = lax.axis_index("c")
    s = lax.axis_index("s")
    wid = c * NS + s
    base_chunk = wid * CPT
    pltpu.sync_copy(src_hbm.at[pl.ds(wid * (CPT // 2), CPT // 2)], srcv)
    pltpu.sync_copy(dst_hbm.at[pl.ds(base_chunk, CPT)], dstv)
    _zero_rows(buf0, 128)
    for k in range(ROWS_PT // 64):
        pltpu.sync_copy(buf0, acc.at[pl.ds(s * ROWS_PT + k * 64, 64)])
    plsc.subcore_barrier()

    pltpu.async_copy(g_hbm.at[srcv.at[0, pl.ds(0, 64)]], buf0, sem0)
    pltpu.async_copy(g_hbm.at[srcv.at[0, pl.ds(64, 64)]], buf1, sem1)

    @pl.loop(0, CPT, step=2)
    def _(j0):
        for b, (buf, sem) in enumerate(((buf0, sem0), (buf1, sem1))):
            j = j0 + b
            pltpu.make_async_copy(g_hbm.at[pl.ds(0, 64)], buf, sem).wait()
            pltpu.sync_copy(buf, acc.at[dstv.at[j]], add=True)

            @pl.when(j + 2 < CPT)
            def _():
                pltpu.async_copy(
                    g_hbm.at[srcv.at[j0 // 2 + 1, pl.ds(b * 64, 64)]],
                    buf, sem)

    plsc.subcore_barrier()
    for k in range(ROWS_PT // 64):
        r = s * ROWS_PT + k * 64
        pltpu.sync_copy(acc.at[pl.ds(r, 64)], out_hbm.at[c, pl.ds(r, 64)])


_scat_kernel = pl.kernel(
    _scat_body,
    out_type=jax.ShapeDtypeStruct((NC, NP, 128), jnp.float32),
    mesh=_mesh,
    scratch_types=[
        pltpu.VMEM((CPT // 2, 128), jnp.int32),
        pltpu.VMEM((CPT, CHUNK), jnp.int32),
        pltpu.VMEM((64, 128), jnp.float32),
        pltpu.VMEM((64, 128), jnp.float32),
        pltpu.VMEM_SHARED((NP, 128), jnp.float32),
        pltpu.SemaphoreType.DMA,
        pltpu.SemaphoreType.DMA,
    ],
)


_R = 512
_NB = NP // _R


def _dinv(d0, d1):
    deg = d0[:, 0:1] + d1[:, 0:1] + 1.0
    return lax.rsqrt(jnp.maximum(deg, 1.0))


def _tc0_body(x_ref, w_ref, d0_ref, d1_ref, g_ref):
    h = jnp.dot(x_ref[:], w_ref[:], preferred_element_type=jnp.float32)
    g_ref[:] = h * _dinv(d0_ref[:], d1_ref[:])


def _tcmid_body(a0_ref, a1_ref, gp_ref, d0_ref, d1_ref, w_ref, b_ref, g_ref):
    dinv = _dinv(d0_ref[:], d1_ref[:])
    f = jnp.tanh((a0_ref[:] + a1_ref[:] + gp_ref[:]) * dinv + b_ref[:])
    h = jnp.dot(f, w_ref[:], preferred_element_type=jnp.float32)
    g_ref[:] = h * dinv


def _tcfinal_body(a0_ref, a1_ref, gp_ref, d0_ref, d1_ref, w_ref, b_ref,
                  bo_ref, o_ref):
    dinv = _dinv(d0_ref[:], d1_ref[:])
    f = jnp.tanh((a0_ref[:] + a1_ref[:] + gp_ref[:]) * dinv + b_ref[:])
    o_ref[:] = jnp.dot(f, w_ref[:], preferred_element_type=jnp.float32) \
        + bo_ref[:]


_row_spec = pl.BlockSpec((_R, 128), lambda i: (i, 0))
_w_spec = pl.BlockSpec((128, 128), lambda i: (0, 0))
_b_spec = pl.BlockSpec((1, 128), lambda i: (0, 0))
_out_struct = jax.ShapeDtypeStruct((NP, 128), jnp.float32)

_tc0 = pl.pallas_call(
    _tc0_body,
    grid=(_NB,),
    in_specs=[_row_spec, _w_spec, _row_spec, _row_spec],
    out_specs=_row_spec,
    out_shape=_out_struct,
)

_tcmid = pl.pallas_call(
    _tcmid_body,
    grid=(_NB,),
    in_specs=[_row_spec, _row_spec, _row_spec, _row_spec, _row_spec,
              _w_spec, _b_spec],
    out_specs=_row_spec,
    out_shape=_out_struct,
)

_tcfinal = pl.pallas_call(
    _tcfinal_body,
    grid=(_NB,),
    in_specs=[_row_spec, _row_spec, _row_spec, _row_spec, _row_spec,
              _w_spec, _b_spec, _b_spec],
    out_specs=_row_spec,
    out_shape=_out_struct,
)


@jax.jit
def _run(x, edge_index, W0, b0, W1, b1, W2, b2, W3, b3, Wo, bo):
    ei = edge_index.astype(jnp.int32)
    pad = jnp.full((EP - E,), NP - 1, jnp.int32)
    src = jnp.concatenate([ei[0], pad]).reshape(EP // 128, 128)
    dst = jnp.concatenate([ei[1], pad]).reshape(EP // CHUNK, CHUNK)
    xp = jnp.pad(x, ((0, NP - N), (0, 0)))
    b0r = b0.reshape(1, 128)
    b1r = b1.reshape(1, 128)
    b2r = b2.reshape(1, 128)
    b3r = b3.reshape(1, 128)
    bor = bo.reshape(1, 128)

    ones = jnp.ones((NP, 128), jnp.float32)
    degs = _scat_kernel(ones, src, dst)
    d0, d1 = degs[0], degs[1]

    g = _tc0(xp, W0, d0, d1)
    acc = _scat_kernel(g, src, dst)
    g = _tcmid(acc[0], acc[1], g, d0, d1, W1, b0r)
    acc = _scat_kernel(g, src, dst)
    g = _tcmid(acc[0], acc[1], g, d0, d1, W2, b1r)
    acc = _scat_kernel(g, src, dst)
    g = _tcmid(acc[0], acc[1], g, d0, d1, W3, b2r)
    acc = _scat_kernel(g, src, dst)
    out = _tcfinal(acc[0], acc[1], g, d0, d1, Wo, b3r, bor)
    return out[:N]


def kernel(x, edge_index, batch, W0, b0, W1, b1, W2, b2, W3, b3, Wo, bo):
    return _run(x, edge_index, W0, b0, W1, b1, W2, b2, W3, b3, Wo, bo)


# speedup vs baseline: 7.1216x; 1.0937x over previous
import jax
import jax.numpy as jnp
from jax import lax
from jax.experimental import pallas as pl
from jax.experimental.pallas import tpu as pltpu
from jax.experimental.pallas import tpu_sc as plsc

N = 10000
E = 320000
NP = 10240
NC = 2
NS = 16
CHUNK = 64
CPT = 160
EPT = CPT * CHUNK
EP = NC * NS * EPT
ROWS_PT = NP // NS

_mesh = plsc.VectorSubcoreMesh(core_axis_name="c", subcore_axis_name="s")


def _zero_rows(ref, width):
    z = jnp.zeros((16,), jnp.float32)

    @pl.loop(0, 64)
    def _(i):
        for k in range(width // 16):
            ref[i, pl.ds(k * 16, 16)] = z


def _scat_body(g_hbm, src_hbm, dst_hbm, out_hbm,
               srcv, dstv, buf0, buf1, acc, gs0, gs1, ss0, ss1):
    c = lax.axis_index("c")
    s = lax.axis_index("s")
    wid = c * NS + s
    base_chunk = wid * CPT
    pltpu.sync_copy(src_hbm.at[pl.ds(wid * (CPT // 2), CPT // 2)], srcv)
    pltpu.sync_copy(dst_hbm.at[pl.ds(base_chunk, CPT)], dstv)
    _zero_rows(buf0, 128)
    for k in range(ROWS_PT // 64):
        pltpu.sync_copy(buf0, acc.at[pl.ds(s * ROWS_PT + k * 64, 64)])
    plsc.subcore_barrier()

    def _wait(sem, buf):
        pltpu.make_async_copy(g_hbm.at[pl.ds(0, 64)], buf, sem).wait()

    def _gather(j, buf, sem):
        pltpu.async_copy(
            g_hbm.at[srcv.at[j // 2, pl.ds((j % 2) * 64, 64)]], buf, sem)

    pltpu.async_copy(g_hbm.at[srcv.at[0, pl.ds(0, 64)]], buf0, gs0)

    @pl.loop(0, CPT, step=2)
    def _(j0):
        for b in range(2):
            j = j0 + b
            buf, gs, ss = (buf0, gs0, ss0) if b == 0 else (buf1, gs1, ss1)
            obuf, ogs, oss = (buf1, gs1, ss1) if b == 0 else (buf0, gs0, ss0)

            @pl.when(j == 0)
            def _():
                _gather(1, obuf, ogs)

            @pl.when(jnp.logical_and(j >= 1, j + 1 < CPT))
            def _():
                _wait(oss, obuf)
                _gather(j + 1, obuf, ogs)

            _wait(gs, buf)
            pltpu.async_copy(buf, acc.at[dstv.at[j]], ss, add=True)

    _wait(ss0, buf0)
    _wait(ss1, buf1)
    plsc.subcore_barrier()
    for k in range(ROWS_PT // 64):
        r = s * ROWS_PT + k * 64
        pltpu.sync_copy(acc.at[pl.ds(r, 64)], out_hbm.at[c, pl.ds(r, 64)])


_scat_kernel = pl.kernel(
    _scat_body,
    out_type=jax.ShapeDtypeStruct((NC, NP, 128), jnp.float32),
    mesh=_mesh,
    scratch_types=[
        pltpu.VMEM((CPT // 2, 128), jnp.int32),
        pltpu.VMEM((CPT, CHUNK), jnp.int32),
        pltpu.VMEM((64, 128), jnp.float32),
        pltpu.VMEM((64, 128), jnp.float32),
        pltpu.VMEM_SHARED((NP, 128), jnp.float32),
        pltpu.SemaphoreType.DMA,
        pltpu.SemaphoreType.DMA,
        pltpu.SemaphoreType.DMA,
        pltpu.SemaphoreType.DMA,
    ],
)


def _deg_body(dst_hbm, out_hbm, dstv, buf0, acc, sem):
    c = lax.axis_index("c")
    s = lax.axis_index("s")
    base_chunk = (c * NS + s) * CPT
    pltpu.sync_copy(dst_hbm.at[pl.ds(base_chunk, CPT)], dstv)
    _zero_rows(buf0, 128)
    for k in range(ROWS_PT // 64):
        pltpu.sync_copy(buf0, acc.at[pl.ds(s * ROWS_PT + k * 64, 64)])
    plsc.subcore_barrier()

    one = jnp.ones((16,), jnp.float32)

    @pl.loop(0, 64)
    def _(i):
        for k in range(8):
            buf0[i, pl.ds(k * 16, 16)] = one

    @pl.loop(0, CPT, step=8)
    def _(j0):
        for b in range(8):
            pltpu.async_copy(buf0, acc.at[dstv.at[j0 + b]], sem, add=True)
        for b in range(8):
            pltpu.make_async_copy(out_hbm.at[c, pl.ds(0, 64)], buf0,
                                  sem).wait()

    plsc.subcore_barrier()
    for k in range(ROWS_PT // 64):
        r = s * ROWS_PT + k * 64
        pltpu.sync_copy(acc.at[pl.ds(r, 64)], out_hbm.at[c, pl.ds(r, 64)])


_deg_kernel = pl.kernel(
    _deg_body,
    out_type=jax.ShapeDtypeStruct((NC, NP, 128), jnp.float32),
    mesh=_mesh,
    scratch_types=[
        pltpu.VMEM((CPT, CHUNK), jnp.int32),
        pltpu.VMEM((64, 128), jnp.float32),
        pltpu.VMEM_SHARED((NP, 128), jnp.float32),
        pltpu.SemaphoreType.DMA,
    ],
)


_R = 512
_NB = NP // _R


def _dinv(d0, d1):
    deg = d0[:, 0:1] + d1[:, 0:1] + 1.0
    return lax.rsqrt(jnp.maximum(deg, 1.0))


def _tc0_body(x_ref, w_ref, d0_ref, d1_ref, g_ref):
    h = jnp.dot(x_ref[:], w_ref[:], preferred_element_type=jnp.float32)
    g_ref[:] = h * _dinv(d0_ref[:], d1_ref[:])


def _tcmid_body(a0_ref, a1_ref, gp_ref, d0_ref, d1_ref, w_ref, b_ref, g_ref):
    dinv = _dinv(d0_ref[:], d1_ref[:])
    f = jnp.tanh((a0_ref[:] + a1_ref[:] + gp_ref[:]) * dinv + b_ref[:])
    h = jnp.dot(f, w_ref[:], preferred_element_type=jnp.float32)
    g_ref[:] = h * dinv


def _tcfinal_body(a0_ref, a1_ref, gp_ref, d0_ref, d1_ref, w_ref, b_ref,
                  bo_ref, o_ref):
    dinv = _dinv(d0_ref[:], d1_ref[:])
    f = jnp.tanh((a0_ref[:] + a1_ref[:] + gp_ref[:]) * dinv + b_ref[:])
    o_ref[:] = jnp.dot(f, w_ref[:], preferred_element_type=jnp.float32) \
        + bo_ref[:]


_row_spec = pl.BlockSpec((_R, 128), lambda i: (i, 0))
_w_spec = pl.BlockSpec((128, 128), lambda i: (0, 0))
_b_spec = pl.BlockSpec((1, 128), lambda i: (0, 0))
_out_struct = jax.ShapeDtypeStruct((NP, 128), jnp.float32)

_tc0 = pl.pallas_call(
    _tc0_body,
    grid=(_NB,),
    in_specs=[_row_spec, _w_spec, _row_spec, _row_spec],
    out_specs=_row_spec,
    out_shape=_out_struct,
)

_tcmid = pl.pallas_call(
    _tcmid_body,
    grid=(_NB,),
    in_specs=[_row_spec, _row_spec, _row_spec, _row_spec, _row_spec,
              _w_spec, _b_spec],
    out_specs=_row_spec,
    out_shape=_out_struct,
)

_tcfinal = pl.pallas_call(
    _tcfinal_body,
    grid=(_NB,),
    in_specs=[_row_spec, _row_spec, _row_spec, _row_spec, _row_spec,
              _w_spec, _b_spec, _b_spec],
    out_specs=_row_spec,
    out_shape=_out_struct,
)


@jax.jit
def _run(x, edge_index, W0, b0, W1, b1, W2, b2, W3, b3, Wo, bo):
    ei = edge_index.astype(jnp.int32)
    pad = jnp.full((EP - E,), NP - 1, jnp.int32)
    src = jnp.concatenate([ei[0], pad]).reshape(EP // 128, 128)
    dst = jnp.concatenate([ei[1], pad]).reshape(EP // CHUNK, CHUNK)
    xp = jnp.pad(x, ((0, NP - N), (0, 0)))
    b0r = b0.reshape(1, 128)
    b1r = b1.reshape(1, 128)
    b2r = b2.reshape(1, 128)
    b3r = b3.reshape(1, 128)
    bor = bo.reshape(1, 128)

    degs = _deg_kernel(dst)
    d0, d1 = degs[0], degs[1]

    g = _tc0(xp, W0, d0, d1)
    acc = _scat_kernel(g, src, dst)
    g = _tcmid(acc[0], acc[1], g, d0, d1, W1, b0r)
    acc = _scat_kernel(g, src, dst)
    g = _tcmid(acc[0], acc[1], g, d0, d1, W2, b1r)
    acc = _scat_kernel(g, src, dst)
    g = _tcmid(acc[0], acc[1], g, d0, d1, W3, b2r)
    acc = _scat_kernel(g, src, dst)
    out = _tcfinal(acc[0], acc[1], g, d0, d1, Wo, b3r, bor)
    return out[:N]


def kernel(x, edge_index, batch, W0, b0, W1, b1, W2, b2, W3, b3, Wo, bo):
    return _run(x, edge_index, W0, b0, W1, b1, W2, b2, W3, b3, Wo, bo)
